# baseline (speedup 1.0000x reference)
import jax
import jax.numpy as jnp
from jax.experimental import pallas as pl
from jax.experimental.pallas import tpu as pltpu

N = 16384
D = 256
K = 1024
BN = 512


def _assign_kernel(x_ref, c_ref, out_ref, cn_ref):
    @pl.when(pl.program_id(0) == 0)
    def _():
        c = c_ref[...]
        cn_ref[...] = jnp.sum(c * c, axis=0, keepdims=True)

    x = x_ref[...]
    prod = jnp.dot(x, c_ref[...], preferred_element_type=jnp.float32)
    scores = cn_ref[...] - 2.0 * prod
    m = jnp.min(scores, axis=-1, keepdims=True)
    idx = jax.lax.broadcasted_iota(jnp.int32, scores.shape, 1)
    am = jnp.min(jnp.where(scores == m, idx, K), axis=-1)
    out_ref[...] = am.reshape(1, 1, BN)


def kernel(x, centroids):
    out = pl.pallas_call(
        _assign_kernel,
        grid=(N // BN,),
        in_specs=[
            pl.BlockSpec((BN, D), lambda i: (i, 0)),
            pl.BlockSpec((D, K), lambda i: (0, 0)),
        ],
        out_specs=pl.BlockSpec((1, 1, BN), lambda i: (i, 0, 0)),
        out_shape=jax.ShapeDtypeStruct((N // BN, 1, BN), jnp.int32),
        scratch_shapes=[pltpu.VMEM((1, K), jnp.float32)],
    )(x, centroids)
    return out.reshape(N)


# speedup vs baseline: 1.6042x; 1.6042x over previous
import jax
import jax.numpy as jnp
from jax.experimental import pallas as pl
from jax.experimental.pallas import tpu as pltpu

N = 16384
D = 256
K = 1024
BN = 512


def _assign_kernel(x_ref, c_ref, out_ref, cn_ref):
    @pl.when(pl.program_id(0) == 0)
    def _():
        c = c_ref[...]
        cn = jnp.sum(c * c, axis=0, keepdims=True)
        cn_ref[...] = jnp.broadcast_to(cn.reshape(K, 1), (K, BN))

    prod_t = jax.lax.dot_general(
        c_ref[...], x_ref[...],
        dimension_numbers=(((0,), (1,)), ((), ())),
        preferred_element_type=jnp.float32)
    scores = cn_ref[...] - 2.0 * prod_t
    am = jnp.argmin(scores, axis=0).astype(jnp.int32)
    out_ref[...] = am.reshape(1, 1, BN)


def kernel(x, centroids):
    out = pl.pallas_call(
        _assign_kernel,
        grid=(N // BN,),
        in_specs=[
            pl.BlockSpec((BN, D), lambda i: (i, 0)),
            pl.BlockSpec((D, K), lambda i: (0, 0)),
        ],
        out_specs=pl.BlockSpec((1, 1, BN), lambda i: (i, 0, 0)),
        out_shape=jax.ShapeDtypeStruct((N // BN, 1, BN), jnp.int32),
        scratch_shapes=[pltpu.VMEM((K, BN), jnp.float32)],
    )(x, centroids)
    return out.reshape(N)


# speedup vs baseline: 2.4030x; 1.4980x over previous
import jax
import jax.numpy as jnp
from jax.experimental import pallas as pl
from jax.experimental.pallas import tpu as pltpu

N = 16384
D = 256
K = 1024
BN = 4096


def _assign_kernel(x_ref, c_ref, out_ref, cn_ref, w_ref):
    @pl.when(pl.program_id(0) == 0)
    def _():
        c = c_ref[...]
        cn = jnp.sum(c * c, axis=0, keepdims=True)
        cn_ref[...] = cn.reshape(K, 1)
        ki = jax.lax.broadcasted_iota(jnp.int32, (8, K), 1)
        si = jax.lax.broadcasted_iota(jnp.int32, (8, K), 0)
        hi = (ki >> 6) << 6
        lo = ki & 63
        w = jnp.where(si == 0, hi, jnp.where(si == 1, lo, 0))
        w_ref[...] = w.astype(jnp.float32)

    prod_t = jax.lax.dot_general(
        c_ref[...], x_ref[...],
        dimension_numbers=(((0,), (1,)), ((), ())),
        preferred_element_type=jnp.float32)
    scores = cn_ref[...] - 2.0 * prod_t
    m = jnp.min(scores, axis=0, keepdims=True)
    mask = jnp.where(scores == m, 1.0, 0.0)
    r = jax.lax.dot_general(
        w_ref[...], mask,
        dimension_numbers=(((1,), (0,)), ((), ())),
        preferred_element_type=jnp.float32)
    idx = (r[0:1, :] + r[1:2, :]).astype(jnp.int32)
    out_ref[...] = idx.reshape(1, 1, BN)


def kernel(x, centroids):
    out = pl.pallas_call(
        _assign_kernel,
        grid=(N // BN,),
        in_specs=[
            pl.BlockSpec((BN, D), lambda i: (i, 0)),
            pl.BlockSpec((D, K), lambda i: (0, 0)),
        ],
        out_specs=pl.BlockSpec((1, 1, BN), lambda i: (i, 0, 0)),
        out_shape=jax.ShapeDtypeStruct((N // BN, 1, BN), jnp.int32),
        scratch_shapes=[pltpu.VMEM((K, 1), jnp.float32),
                        pltpu.VMEM((8, K), jnp.float32)],
    )(x, centroids)
    return out.reshape(N)


# speedup vs baseline: 2.4635x; 1.0252x over previous
import jax
import jax.numpy as jnp
from jax.experimental import pallas as pl
from jax.experimental.pallas import tpu as pltpu

N = 16384
D = 256
K = 1024
BN = 4096
SB = 2048


def _assign_kernel(x_ref, c_ref, out_ref, cn_ref):
    @pl.when(pl.program_id(0) == 0)
    def _():
        c = c_ref[...]
        cn = jnp.sum(c * c, axis=0, keepdims=True)
        cn_ref[...] = cn.reshape(K, 1)

    for j in range(BN // SB):
        xj = x_ref[pl.ds(j * SB, SB), :]
        prod_t = jax.lax.dot_general(
            c_ref[...], xj,
            dimension_numbers=(((0,), (1,)), ((), ())),
            preferred_element_type=jnp.float32)
        scores = cn_ref[...] - 2.0 * prod_t
        am = jnp.argmin(scores, axis=0).astype(jnp.int32)
        out_ref[0, 0, pl.ds(j * SB, SB)] = am


def kernel(x, centroids):
    out = pl.pallas_call(
        _assign_kernel,
        grid=(N // BN,),
        in_specs=[
            pl.BlockSpec((BN, D), lambda i: (i, 0)),
            pl.BlockSpec((D, K), lambda i: (0, 0)),
        ],
        out_specs=pl.BlockSpec((1, 1, BN), lambda i: (i, 0, 0)),
        out_shape=jax.ShapeDtypeStruct((N // BN, 1, BN), jnp.int32),
        scratch_shapes=[pltpu.VMEM((K, 1), jnp.float32)],
    )(x, centroids)
    return out.reshape(N)
